# Initial kernel scaffold; baseline (speedup 1.0000x reference)
import jax
import jax.numpy as jnp
from jax.experimental import pallas as pl


def kernel(x, edge_index0, edge_index1, n_dst0, n_dst1, W0_l, W0_r, bn0_g, bn0_b, res0_W, res0_b, W1_l, W1_r, bn1_g, bn1_b, mlp_W1, mlp_b1, mlp_W2, mlp_b2):
    raise NotImplementedError("write your pallas kernel here")


# speedup vs baseline: 6.8192x; 6.8192x over previous
import dataclasses
import functools

import jax
import jax.numpy as jnp
from jax import lax
from jax.experimental import pallas as pl
from jax.experimental.pallas import tpu as pltpu
from jax.experimental.pallas import tpu_sc as plsc

F = 128
NDST = 10240
CH = 128
NC = 2
NS = 16
NW = NC * NS
RSTRIPE = NDST // NS
HR = NDST // F

_CP = pltpu.CompilerParams()
if "needs_layout_passes" in pltpu.CompilerParams.__dataclass_fields__:
    _CP = dataclasses.replace(_CP, needs_layout_passes=False)


def _make_agg(chunks):
    per_tile = chunks * CH
    mesh = plsc.VectorSubcoreMesh(core_axis_name="c", subcore_axis_name="s")

    @functools.partial(
        pl.kernel,
        mesh=mesh,
        compiler_params=_CP,
        out_type=(
            jax.ShapeDtypeStruct((NC * NDST, F), jnp.float32),
            jax.ShapeDtypeStruct((NC * HR, F), jnp.float32),
        ),
        scratch_types=[
            pltpu.VMEM_SHARED((NDST, F), jnp.float32),
            pltpu.VMEM_SHARED((HR, F), jnp.float32),
            pltpu.VMEM((HR, F), jnp.float32),
            pltpu.VMEM((HR,), jnp.int32),
            pltpu.VMEM((CH,), jnp.int32),
            pltpu.VMEM((CH,), jnp.int32),
            pltpu.VMEM((CH, F), jnp.float32),
            pltpu.SemaphoreType.DMA,
        ],
    )
    def agg(table, src_hbm, dst_hbm, sum_out, cnt_out,
            acc, cntacc, hist, idxb, srcb, dstb, rows, sem):
        cid = lax.axis_index("c")
        sid = lax.axis_index("s")
        wid = cid * NS + sid
        z16 = jnp.zeros((16,), jnp.float32)
        one16 = jnp.full((16,), 1.0, jnp.float32)
        iota16 = lax.iota(jnp.int32, 16)

        @pl.loop(0, HR // 16)
        def _(g):
            idxb[pl.ds(g * 16, 16)] = iota16 + g * 16

        @pl.loop(0, CH)
        def _(i):
            @pl.loop(0, F // 16)
            def _(j):
                rows[i, pl.ds(j * 16, 16)] = z16

        @pl.loop(0, HR)
        def _(i):
            @pl.loop(0, F // 16)
            def _(j):
                hist[i, pl.ds(j * 16, 16)] = z16

        @pl.loop(0, RSTRIPE // CH)
        def _(j):
            r0 = sid * RSTRIPE + j * CH
            pltpu.sync_copy(rows, acc.at[pl.ds(r0, CH)])

        pltpu.sync_copy(hist.at[pl.ds(0, HR // NS)],
                        cntacc.at[pl.ds(sid * (HR // NS), HR // NS)])
        plsc.subcore_barrier()
        base = wid * per_tile

        @pl.loop(0, chunks)
        def _(k):
            off = base + k * CH
            pltpu.sync_copy(src_hbm.at[pl.ds(off, CH)], srcb)
            pltpu.sync_copy(dst_hbm.at[pl.ds(off, CH)], dstb)
            pltpu.async_copy(
                table.at[plsc.Indices(srcb, ignored_value=-1)], rows, sem
            ).wait()
            pltpu.sync_copy(
                rows, acc.at[plsc.Indices(dstb, ignored_value=-1)], add=True
            )

            @pl.loop(0, CH // 16)
            def _(g):
                idx16 = dstb[pl.ds(g * 16, 16)]
                valid = idx16 >= 0
                row = lax.shift_right_logical(jnp.maximum(idx16, 0), 7)
                col = lax.bitwise_and(idx16, 127)
                plsc.addupdate_scatter(hist, [row, col], one16, mask=valid)

        pltpu.sync_copy(hist, cntacc.at[plsc.Indices(idxb)], add=True)
        plsc.subcore_barrier()

        @pl.loop(0, RSTRIPE // CH)
        def _(j):
            r0 = sid * RSTRIPE + j * CH
            pltpu.sync_copy(acc.at[pl.ds(r0, CH)], rows)
            pltpu.sync_copy(rows, sum_out.at[pl.ds(cid * NDST + r0, CH)])

        @pl.when(sid == 0)
        def _():
            pltpu.sync_copy(cntacc.at[pl.ds(0, HR)], hist)
            pltpu.sync_copy(hist, cnt_out.at[pl.ds(cid * HR, HR)])

    return agg


RB = 2000
GRID = 10000 // RB
_BN_SCALE = 1.0 / (1.0 + 1e-5) ** 0.5


def _leaky(v):
    return jnp.where(v >= 0, v, 0.01 * v)


def _tc1_body(sum_ref, cnt_ref, x_ref, w0l, w0r, resw, g0, b0, rb0,
              h_ref, hr_ref):
    s = sum_ref[0] + sum_ref[1]
    c = cnt_ref[0] + cnt_ref[1]
    mean = s / jnp.maximum(c, 1.0)
    x = x_ref[...]
    pre = (jnp.dot(mean, w0l[...], preferred_element_type=jnp.float32)
           + jnp.dot(x, w0r[...], preferred_element_type=jnp.float32))
    pre = pre * (g0[...] * _BN_SCALE) + b0[...]
    h = _leaky(pre)
    h_ref[...] = h
    hr_ref[...] = h + jnp.dot(x, resw[...],
                              preferred_element_type=jnp.float32) + rb0[...]


def _tc2_body(sum_ref, cnt_ref, hr_ref, x_ref, h_ref, w1l, w1r, g1, b1,
              w1x, w1h, w1h2, mb1, mw2, mb2, out_ref):
    s = sum_ref[0] + sum_ref[1]
    c = cnt_ref[0] + cnt_ref[1]
    mean = s / jnp.maximum(c, 1.0)
    hr = hr_ref[...]
    pre = (jnp.dot(mean, w1l[...], preferred_element_type=jnp.float32)
           + jnp.dot(hr, w1r[...], preferred_element_type=jnp.float32))
    pre = pre * (g1[...] * _BN_SCALE) + b1[...]
    h2 = _leaky(pre)
    z1 = (jnp.dot(x_ref[...], w1x[...], preferred_element_type=jnp.float32)
          + jnp.dot(h_ref[...], w1h[...], preferred_element_type=jnp.float32)
          + jnp.dot(h2, w1h2[...], preferred_element_type=jnp.float32)
          + mb1[...])
    z = jnp.dot(z1, mw2[...], preferred_element_type=jnp.float32) + mb2[...]
    m = jnp.max(z, axis=-1, keepdims=True)
    lse = m + jnp.log(jnp.sum(jnp.exp(z - m), axis=-1, keepdims=True))
    out_ref[...] = z - lse


def _full(shape):
    return pl.BlockSpec(shape, lambda i: tuple(0 for _ in shape))


def _rows(shape):
    return pl.BlockSpec(shape, lambda i: (i,) + tuple(0 for _ in shape[1:]))


def _mid(shape):
    return pl.BlockSpec(shape, lambda i: (0, i) + tuple(0 for _ in shape[2:]))


def kernel(x, edge_index0, edge_index1, n_dst0, n_dst1, W0_l, W0_r, bn0_g,
           bn0_b, res0_W, res0_b, W1_l, W1_r, bn1_g, bn1_b, mlp_W1, mlp_b1,
           mlp_W2, mlp_b2):
    x = x.astype(jnp.float32)
    e0 = edge_index0.shape[1]
    e1 = edge_index1.shape[1]
    chunks0 = -(-e0 // (NW * CH))
    chunks1 = -(-e1 // (NW * CH))
    np0 = chunks0 * NW * CH
    np1 = chunks1 * NW * CH

    ei0 = edge_index0.astype(jnp.int32)
    ei1 = edge_index1.astype(jnp.int32)
    keep0 = ei0[1] < 10000
    src0 = jnp.pad(jnp.where(keep0, ei0[0], -1), (0, np0 - e0),
                   constant_values=-1)
    dst0 = jnp.pad(jnp.where(keep0, ei0[1], -1), (0, np0 - e0),
                   constant_values=-1)
    src1 = jnp.pad(ei1[0], (0, np1 - e1), constant_values=-1)
    dst1 = jnp.pad(ei1[1], (0, np1 - e1), constant_values=-1)

    sum0, cnt0 = _make_agg(chunks0)(x, src0, dst0)
    sum0 = sum0.reshape(NC, NDST, F)
    cnt0 = cnt0.reshape(NC, NDST, 1)

    g0 = bn0_g.reshape(1, F)
    b0 = bn0_b.reshape(1, F)
    rb0 = res0_b.reshape(1, F)
    h, hr = pl.pallas_call(
        _tc1_body,
        grid=(GRID,),
        in_specs=[
            _mid((NC, RB, F)), _mid((NC, RB, 1)), _rows((RB, F)),
            _full((F, F)), _full((F, F)), _full((F, F)),
            _full((1, F)), _full((1, F)), _full((1, F)),
        ],
        out_specs=[_rows((RB, F)), _rows((RB, F))],
        out_shape=[jax.ShapeDtypeStruct((10000, F), jnp.float32)] * 2,
    )(sum0, cnt0, x, W0_l, W0_r, res0_W, g0, b0, rb0)

    sum1, cnt1 = _make_agg(chunks1)(hr, src1, dst1)
    sum1 = sum1.reshape(NC, NDST, F)
    cnt1 = cnt1.reshape(NC, NDST, 1)

    g1 = bn1_g.reshape(1, F)
    b1 = bn1_b.reshape(1, F)
    w1x = mlp_W1[:F]
    w1h = mlp_W1[F:2 * F]
    w1h2 = mlp_W1[2 * F:]
    mb1 = mlp_b1.reshape(1, -1)
    mb2 = mlp_b2.reshape(1, -1)
    out = pl.pallas_call(
        _tc2_body,
        grid=(GRID,),
        in_specs=[
            _mid((NC, RB, F)), _mid((NC, RB, 1)), _rows((RB, F)),
            _rows((RB, F)), _rows((RB, F)),
            _full((F, F)), _full((F, F)), _full((1, F)), _full((1, F)),
            _full((F, 2 * F)), _full((F, 2 * F)), _full((F, 2 * F)),
            _full((1, 2 * F)), _full((2 * F, F)), _full((1, F)),
        ],
        out_specs=[_rows((RB, F))],
        out_shape=[jax.ShapeDtypeStruct((10000, F), jnp.float32)],
    )(sum1, cnt1, hr, x, h, W1_l, W1_r, g1, b1, w1x, w1h, w1h2, mb1,
      mlp_W2, mb2)[0]
    return out


# speedup vs baseline: 11.1887x; 1.6408x over previous
import dataclasses
import functools

import jax
import jax.numpy as jnp
from jax import lax
from jax.experimental import pallas as pl
from jax.experimental.pallas import tpu as pltpu
from jax.experimental.pallas import tpu_sc as plsc

F = 128
NDST = 10240
CH = 128
NC = 2
NS = 16
NW = NC * NS
RSTRIPE = NDST // NS
HR = NDST // F

_CP = pltpu.CompilerParams()
if "needs_layout_passes" in pltpu.CompilerParams.__dataclass_fields__:
    _CP = dataclasses.replace(_CP, needs_layout_passes=False)


NBUF = 2
NIDX = 4


def _make_agg(chunks):
    mesh = plsc.VectorSubcoreMesh(core_axis_name="c", subcore_axis_name="s")

    @functools.partial(
        pl.kernel,
        mesh=mesh,
        compiler_params=_CP,
        out_type=(
            jax.ShapeDtypeStruct((NC * NDST, F), jnp.float32),
            jax.ShapeDtypeStruct((NC * HR, F), jnp.float32),
        ),
        scratch_types=[
            pltpu.VMEM_SHARED((NDST, F), jnp.float32),
            pltpu.VMEM_SHARED((HR, F), jnp.float32),
            pltpu.VMEM((HR, F), jnp.float32),
            pltpu.VMEM((HR,), jnp.int32),
        ] + [pltpu.VMEM((2, CH), jnp.int32) for _ in range(NIDX)]
          + [pltpu.VMEM((CH, F), jnp.float32) for _ in range(NBUF)]
          + [pltpu.SemaphoreType.DMA for _ in range(NIDX + 2 * NBUF)],
    )
    def agg(table, packed_hbm, sum_out, cnt_out,
            acc, cntacc, hist, idxb, *rest):
        ebch = rest[:NIDX]
        rowsl = rest[NIDX:NIDX + NBUF]
        si = rest[NIDX + NBUF:2 * NIDX + NBUF]
        sg = rest[2 * NIDX + NBUF:2 * NIDX + 2 * NBUF]
        ss = rest[2 * NIDX + 2 * NBUF:]
        cid = lax.axis_index("c")
        sid = lax.axis_index("s")
        wid = cid * NS + sid
        z16 = jnp.zeros((16,), jnp.float32)
        one16 = jnp.full((16,), 1.0, jnp.float32)
        iota16 = lax.iota(jnp.int32, 16)
        base = wid * chunks

        def start_idx(j, k):
            pltpu.async_copy(packed_hbm.at[base + k], ebch[j], si[j])

        def wait_idx(j):
            pltpu.make_async_copy(packed_hbm.at[0], ebch[j], si[j]).wait()

        def start_gather(b, j):
            pltpu.async_copy(
                table.at[plsc.Indices(ebch[j].at[0], ignored_value=-1)],
                rowsl[b], sg[b])

        def wait_gather(b):
            pltpu.make_async_copy(
                table.at[pl.ds(0, CH)], rowsl[b], sg[b]).wait()

        def start_scatter(b, j):
            pltpu.async_copy(
                rowsl[b],
                acc.at[plsc.Indices(ebch[j].at[1], ignored_value=-1)],
                ss[b], add=True)

        def wait_scatter(b):
            pltpu.make_async_copy(
                rowsl[b], sum_out.at[pl.ds(0, CH)], ss[b]).wait()

        def hist_update(j):
            @pl.loop(0, CH // 16)
            def _(g):
                idx16 = ebch[j][1, pl.ds(g * 16, 16)]
                valid = idx16 >= 0
                row = lax.shift_right_logical(jnp.maximum(idx16, 0), 7)
                col = lax.bitwise_and(idx16, 127)
                plsc.addupdate_scatter(hist, [row, col], one16, mask=valid)

        for j in range(NIDX):
            start_idx(j, j)

        @pl.loop(0, HR // 16)
        def _(g):
            idxb[pl.ds(g * 16, 16)] = iota16 + g * 16

        @pl.loop(0, CH)
        def _(i):
            @pl.loop(0, F // 16)
            def _(j):
                rowsl[0][i, pl.ds(j * 16, 16)] = z16

        @pl.loop(0, HR)
        def _(i):
            @pl.loop(0, F // 16)
            def _(j):
                hist[i, pl.ds(j * 16, 16)] = z16

        @pl.loop(0, RSTRIPE // CH)
        def _(j):
            r0 = sid * RSTRIPE + j * CH
            pltpu.sync_copy(rowsl[0], acc.at[pl.ds(r0, CH)])

        pltpu.sync_copy(hist.at[pl.ds(0, HR // NS)],
                        cntacc.at[pl.ds(sid * (HR // NS), HR // NS)])
        plsc.subcore_barrier()

        wait_idx(0)
        start_gather(0, 0)
        wait_idx(1)
        start_gather(1, 1)

        @pl.loop(0, chunks // 4 - 1)
        def _(kk):
            k0 = 4 * kk
            wait_gather(0)
            start_scatter(0, 0)
            hist_update(0)
            wait_gather(1)
            start_scatter(1, 1)
            hist_update(1)
            wait_scatter(0)
            wait_idx(2)
            start_gather(0, 2)
            start_idx(0, k0 + 4)
            wait_scatter(1)
            wait_idx(3)
            start_gather(1, 3)
            start_idx(1, k0 + 5)
            wait_gather(0)
            start_scatter(0, 2)
            hist_update(2)
            wait_gather(1)
            start_scatter(1, 3)
            hist_update(3)
            wait_scatter(0)
            wait_idx(0)
            start_gather(0, 0)
            start_idx(2, k0 + 6)
            wait_scatter(1)
            wait_idx(1)
            start_gather(1, 1)
            start_idx(3, k0 + 7)

        wait_gather(0)
        start_scatter(0, 0)
        hist_update(0)
        wait_gather(1)
        start_scatter(1, 1)
        hist_update(1)
        wait_scatter(0)
        wait_idx(2)
        start_gather(0, 2)
        wait_scatter(1)
        wait_idx(3)
        start_gather(1, 3)
        wait_gather(0)
        start_scatter(0, 2)
        hist_update(2)
        wait_gather(1)
        start_scatter(1, 3)
        hist_update(3)
        wait_scatter(0)
        wait_scatter(1)

        pltpu.sync_copy(hist, cntacc.at[plsc.Indices(idxb)], add=True)
        plsc.subcore_barrier()

        @pl.loop(0, RSTRIPE // CH)
        def _(j):
            r0 = sid * RSTRIPE + j * CH
            pltpu.sync_copy(acc.at[pl.ds(r0, CH)], rowsl[0])
            pltpu.sync_copy(rowsl[0], sum_out.at[pl.ds(cid * NDST + r0, CH)])

        @pl.when(sid == 0)
        def _():
            pltpu.sync_copy(cntacc.at[pl.ds(0, HR)], hist)
            pltpu.sync_copy(hist, cnt_out.at[pl.ds(cid * HR, HR)])

    return agg


RB = 2000
GRID = 10000 // RB
_BN_SCALE = 1.0 / (1.0 + 1e-5) ** 0.5


def _leaky(v):
    return jnp.where(v >= 0, v, 0.01 * v)


def _tc1_body(sum_ref, cnt_ref, x_ref, w0l, w0r, resw, g0, b0, rb0,
              h_ref, hr_ref):
    s = sum_ref[0] + sum_ref[1]
    c = cnt_ref[0] + cnt_ref[1]
    mean = s / jnp.maximum(c, 1.0)
    x = x_ref[...]
    pre = (jnp.dot(mean, w0l[...], preferred_element_type=jnp.float32)
           + jnp.dot(x, w0r[...], preferred_element_type=jnp.float32))
    pre = pre * (g0[...] * _BN_SCALE) + b0[...]
    h = _leaky(pre)
    h_ref[...] = h
    hr_ref[...] = h + jnp.dot(x, resw[...],
                              preferred_element_type=jnp.float32) + rb0[...]


def _tc2_body(sum_ref, cnt_ref, hr_ref, x_ref, h_ref, w1l, w1r, g1, b1,
              w1x, w1h, w1h2, mb1, mw2, mb2, out_ref):
    s = sum_ref[0] + sum_ref[1]
    c = cnt_ref[0] + cnt_ref[1]
    mean = s / jnp.maximum(c, 1.0)
    hr = hr_ref[...]
    pre = (jnp.dot(mean, w1l[...], preferred_element_type=jnp.float32)
           + jnp.dot(hr, w1r[...], preferred_element_type=jnp.float32))
    pre = pre * (g1[...] * _BN_SCALE) + b1[...]
    h2 = _leaky(pre)
    z1 = (jnp.dot(x_ref[...], w1x[...], preferred_element_type=jnp.float32)
          + jnp.dot(h_ref[...], w1h[...], preferred_element_type=jnp.float32)
          + jnp.dot(h2, w1h2[...], preferred_element_type=jnp.float32)
          + mb1[...])
    z = jnp.dot(z1, mw2[...], preferred_element_type=jnp.float32) + mb2[...]
    m = jnp.max(z, axis=-1, keepdims=True)
    lse = m + jnp.log(jnp.sum(jnp.exp(z - m), axis=-1, keepdims=True))
    out_ref[...] = z - lse


def _full(shape):
    return pl.BlockSpec(shape, lambda i: tuple(0 for _ in shape))


def _rows(shape):
    return pl.BlockSpec(shape, lambda i: (i,) + tuple(0 for _ in shape[1:]))


def _mid(shape):
    return pl.BlockSpec(shape, lambda i: (0, i) + tuple(0 for _ in shape[2:]))


def kernel(x, edge_index0, edge_index1, n_dst0, n_dst1, W0_l, W0_r, bn0_g,
           bn0_b, res0_W, res0_b, W1_l, W1_r, bn1_g, bn1_b, mlp_W1, mlp_b1,
           mlp_W2, mlp_b2):
    x = x.astype(jnp.float32)
    e0 = edge_index0.shape[1]
    e1 = edge_index1.shape[1]
    chunks0 = -(-(-(-e0 // (NW * CH))) // NBUF) * NBUF
    chunks1 = -(-(-(-e1 // (NW * CH))) // NBUF) * NBUF
    np0 = chunks0 * NW * CH
    np1 = chunks1 * NW * CH

    ei0 = edge_index0.astype(jnp.int32)
    ei1 = edge_index1.astype(jnp.int32)
    keep0 = ei0[1] < 10000
    src0 = jnp.pad(jnp.where(keep0, ei0[0], -1), (0, np0 - e0),
                   constant_values=-1)
    dst0 = jnp.pad(jnp.where(keep0, ei0[1], -1), (0, np0 - e0),
                   constant_values=-1)
    src1 = jnp.pad(ei1[0], (0, np1 - e1), constant_values=-1)
    dst1 = jnp.pad(ei1[1], (0, np1 - e1), constant_values=-1)

    def _pack(src, dst, chunks):
        return jnp.stack([src.reshape(NW, chunks, CH),
                          dst.reshape(NW, chunks, CH)],
                         axis=2).reshape(NW * chunks, 2, CH)

    packed0 = _pack(src0, dst0, chunks0)
    packed1 = _pack(src1, dst1, chunks1)

    sum0, cnt0 = _make_agg(chunks0)(x, packed0)
    sum0 = sum0.reshape(NC, NDST, F)
    cnt0 = cnt0.reshape(NC, NDST, 1)

    g0 = bn0_g.reshape(1, F)
    b0 = bn0_b.reshape(1, F)
    rb0 = res0_b.reshape(1, F)
    h, hr = pl.pallas_call(
        _tc1_body,
        grid=(GRID,),
        in_specs=[
            _mid((NC, RB, F)), _mid((NC, RB, 1)), _rows((RB, F)),
            _full((F, F)), _full((F, F)), _full((F, F)),
            _full((1, F)), _full((1, F)), _full((1, F)),
        ],
        out_specs=[_rows((RB, F)), _rows((RB, F))],
        out_shape=[jax.ShapeDtypeStruct((10000, F), jnp.float32)] * 2,
    )(sum0, cnt0, x, W0_l, W0_r, res0_W, g0, b0, rb0)

    sum1, cnt1 = _make_agg(chunks1)(hr, packed1)
    sum1 = sum1.reshape(NC, NDST, F)
    cnt1 = cnt1.reshape(NC, NDST, 1)

    g1 = bn1_g.reshape(1, F)
    b1 = bn1_b.reshape(1, F)
    w1x = mlp_W1[:F]
    w1h = mlp_W1[F:2 * F]
    w1h2 = mlp_W1[2 * F:]
    mb1 = mlp_b1.reshape(1, -1)
    mb2 = mlp_b2.reshape(1, -1)
    out = pl.pallas_call(
        _tc2_body,
        grid=(GRID,),
        in_specs=[
            _mid((NC, RB, F)), _mid((NC, RB, 1)), _rows((RB, F)),
            _rows((RB, F)), _rows((RB, F)),
            _full((F, F)), _full((F, F)), _full((1, F)), _full((1, F)),
            _full((F, 2 * F)), _full((F, 2 * F)), _full((F, 2 * F)),
            _full((1, 2 * F)), _full((2 * F, F)), _full((1, F)),
        ],
        out_specs=[_rows((RB, F))],
        out_shape=[jax.ShapeDtypeStruct((10000, F), jnp.float32)],
    )(sum1, cnt1, hr, x, h, W1_l, W1_r, g1, b1, w1x, w1h, w1h2, mb1,
      mlp_W2, mb2)[0]
    return out
